# speedup vs baseline: 1.4949x; 1.4949x over previous
import functools

import jax
import jax.numpy as jnp
from jax import lax
from jax.experimental import pallas as pl
from jax.experimental.pallas import tpu as pltpu
from jax.experimental.pallas import tpu_sc as plsc

DIM = 128
SEQ_TOTAL = 8192
CHUNK = 128
NUM_CHUNKS = SEQ_TOTAL // CHUNK

_info = plsc.get_sparse_core_info()
_NC, _NS = _info.num_cores, _info.num_subcores
_NW = _NC * _NS
CPW = NUM_CHUNKS // _NW

_mesh = plsc.VectorSubcoreMesh(core_axis_name="c", subcore_axis_name="s")


@functools.partial(
    pl.kernel,
    mesh=_mesh,
    out_type=(
        jax.ShapeDtypeStruct((NUM_CHUNKS, CHUNK, DIM), jnp.float32),
        jax.ShapeDtypeStruct((NUM_CHUNKS, CHUNK, DIM), jnp.float32),
    ),
    scratch_types=[
        pltpu.VMEM((CPW, CHUNK), jnp.int32),
        pltpu.VMEM((CPW, CHUNK, DIM), jnp.float32),
        pltpu.VMEM((CPW, CHUNK, DIM), jnp.float32),
        pltpu.SemaphoreType.DMA,
        pltpu.SemaphoreType.DMA,
    ],
)
def _rope_gather(cos_hbm, sin_hbm, idx_hbm, cos_out, sin_out,
                 idx_v, cos_v, sin_v, gsem, ssem):
    wid = lax.axis_index("s") * _NC + lax.axis_index("c")
    base = wid * CPW
    pltpu.sync_copy(idx_hbm.at[pl.ds(base, CPW)], idx_v)
    copies = []
    for j in range(CPW):
        copies.append(pltpu.async_copy(cos_hbm.at[idx_v.at[j]], cos_v.at[j], gsem))
        copies.append(pltpu.async_copy(sin_hbm.at[idx_v.at[j]], sin_v.at[j], gsem))
    for c in copies:
        c.wait()
    stores = [
        pltpu.async_copy(cos_v, cos_out.at[pl.ds(base, CPW)], ssem),
        pltpu.async_copy(sin_v, sin_out.at[pl.ds(base, CPW)], ssem),
    ]
    for c in stores:
        c.wait()


def kernel(x, position_ids, cos_cached, sin_cached):
    idx = position_ids.astype(jnp.int32).reshape(NUM_CHUNKS, CHUNK)
    cos, sin = _rope_gather(cos_cached, sin_cached, idx)
    b, s = position_ids.shape
    return (cos.reshape(b, 1, s, DIM), sin.reshape(b, 1, s, DIM))


# speedup vs baseline: 1.4970x; 1.0014x over previous
import functools

import jax
import jax.numpy as jnp
from jax import lax
from jax.experimental import pallas as pl
from jax.experimental.pallas import tpu as pltpu
from jax.experimental.pallas import tpu_sc as plsc

DIM = 128
SEQ_TOTAL = 8192
CHUNK = 128
NUM_CHUNKS = SEQ_TOTAL // CHUNK

_info = plsc.get_sparse_core_info()
_NC, _NS = _info.num_cores, _info.num_subcores
_NW = _NC * _NS
CPW = NUM_CHUNKS // _NW

_mesh = plsc.VectorSubcoreMesh(core_axis_name="c", subcore_axis_name="s")


@functools.partial(
    pl.kernel,
    mesh=_mesh,
    out_type=(
        jax.ShapeDtypeStruct((NUM_CHUNKS, CHUNK, DIM), jnp.float32),
        jax.ShapeDtypeStruct((NUM_CHUNKS, CHUNK, DIM), jnp.float32),
    ),
    scratch_types=[
        pltpu.VMEM((CPW, CHUNK), jnp.int32),
        pltpu.VMEM((CPW, CHUNK, DIM), jnp.float32),
        pltpu.VMEM((CPW, CHUNK, DIM), jnp.float32),
        pltpu.SemaphoreType.DMA,
        pltpu.SemaphoreType.DMA,
        pltpu.SemaphoreType.DMA,
    ],
)
def _rope_gather(cos_hbm, sin_hbm, idx_hbm, cos_out, sin_out,
                 idx_v, cos_v, sin_v, gsem0, gsem1, ssem):
    wid = lax.axis_index("s") * _NC + lax.axis_index("c")
    base = wid * CPW
    pltpu.sync_copy(idx_hbm.at[pl.ds(base, CPW)], idx_v)
    gsems = (gsem0, gsem1)
    gathers = []
    for j in range(CPW):
        gathers.append((
            pltpu.async_copy(cos_hbm.at[idx_v.at[j]], cos_v.at[j], gsems[j]),
            pltpu.async_copy(sin_hbm.at[idx_v.at[j]], sin_v.at[j], gsems[j]),
        ))
    stores = []
    for j in range(CPW):
        gc, gs = gathers[j]
        gc.wait()
        gs.wait()
        stores.append(pltpu.async_copy(
            cos_v.at[pl.ds(j, 1)], cos_out.at[pl.ds(base + j, 1)], ssem))
        stores.append(pltpu.async_copy(
            sin_v.at[pl.ds(j, 1)], sin_out.at[pl.ds(base + j, 1)], ssem))
    for c in stores:
        c.wait()


def kernel(x, position_ids, cos_cached, sin_cached):
    idx = position_ids.astype(jnp.int32).reshape(NUM_CHUNKS, CHUNK)
    cos, sin = _rope_gather(cos_cached, sin_cached, idx)
    b, s = position_ids.shape
    return (cos.reshape(b, 1, s, DIM), sin.reshape(b, 1, s, DIM))
